# baseline (speedup 1.0000x reference)
import functools

import jax
import jax.numpy as jnp
from jax.experimental import pallas as pl
from jax.experimental.pallas import tpu as pltpu
from jax.experimental.pallas import tpu_sc as plsc

_B, _C = 4096, 1000
_CP = 1024
_BR = 256
_NBLK = _B // _BR
_TK_W, _G_W, _HN_W, _HN_K = 0.6, 0.4, 0.5, 2
_TOPK = 6

_NC, _NS = 2, 16
_NW = _NC * _NS
_RPT = _B // _NW
_NRC = 16
_NCH = _RPT // _NRC

_NLN2 = -0.6931471805599453


def _bce(p, t):
    p = jnp.clip(p, 1e-7, 1.0 - 1e-7)
    l2p = jnp.log2(p)
    l2q = jnp.log2(1.0 - p)
    return _NLN2 * (l2q + t * (l2p - l2q))


_sc_mesh = plsc.VectorSubcoreMesh(core_axis_name="c", subcore_axis_name="s")


@functools.partial(
    pl.kernel,
    mesh=_sc_mesh,
    out_type=[jax.ShapeDtypeStruct((_B * 16,), jnp.float32) for _ in range(4)],
    scratch_types=[
        pltpu.VMEM((_NRC * _CP,), jnp.float32),
        pltpu.VMEM((_NRC * _CP,), jnp.float32),
        pltpu.VMEM((_NRC * 16,), jnp.float32),
        pltpu.VMEM((_NRC * 16,), jnp.float32),
        pltpu.SemaphoreType.DMA,
    ],
    compiler_params=pltpu.CompilerParams(needs_layout_passes=False),
)
def _sc_topk(tks, gs, tkt, gt, otkv, otkt, ogv, ogt, sbuf, tbuf, obv, obt, sem):
    wid = jax.lax.axis_index("s") * _NC + jax.lax.axis_index("c")
    iota = jax.lax.iota(jnp.int32, 16)
    i16 = iota * 16
    z16 = iota * 0
    neg1 = jnp.full((16,), -1.0, jnp.float32)
    lane0 = iota == 0

    def bcast0(x):
        return x.at[z16].get(mode="promise_in_bounds")

    for r in range(_NRC):
        sbuf[pl.ds(r * _CP + 992, 16)] = neg1
        sbuf[pl.ds(r * _CP + 1008, 16)] = neg1

    for s_hbm, t_hbm, ov_hbm, ot_hbm in ((tks, tkt, otkv, otkt),
                                         (gs, gt, ogv, ogt)):
        def chunk_body(ch, carry, s_hbm=s_hbm, t_hbm=t_hbm,
                       ov_hbm=ov_hbm, ot_hbm=ot_hbm):
            row0 = wid * _RPT + ch * _NRC
            copies = []
            for r in range(_NRC):
                copies.append(pltpu.async_copy(
                    s_hbm.at[pl.ds((row0 + r) * _C, _C)],
                    sbuf.at[pl.ds(r * _CP, _C)], sem))
                copies.append(pltpu.async_copy(
                    t_hbm.at[pl.ds((row0 + r) * _C, _C)],
                    tbuf.at[pl.ds(r * _CP, _C)], sem))
            for cp in copies:
                cp.wait()

            def row_body(r, carry2):
                rb = r * _CP
                a0 = sbuf[pl.ds(rb, 16)]
                a1 = sbuf[pl.ds(rb + 16, 16)]
                a2 = sbuf[pl.ds(rb + 32, 16)]
                a3 = sbuf[pl.ds(rb + 48, 16)]
                for v in range(4, 64, 4):
                    a0 = jnp.maximum(a0, sbuf[pl.ds(rb + v * 16, 16)])
                    a1 = jnp.maximum(a1, sbuf[pl.ds(rb + v * 16 + 16, 16)])
                    a2 = jnp.maximum(a2, sbuf[pl.ds(rb + v * 16 + 32, 16)])
                    a3 = jnp.maximum(a3, sbuf[pl.ds(rb + v * 16 + 48, 16)])
                M = jnp.maximum(jnp.maximum(a0, a1), jnp.maximum(a2, a3))
                rbv = jnp.full((16,), rb, jnp.int32)
                res_v = jnp.zeros((16,), jnp.float32)
                res_i = rbv
                for e in range(_TOPK):
                    svals, sids = plsc.sort_key_val(M, iota, descending=True)
                    m = bcast0(svals)
                    l = bcast0(sids)
                    base = rbv + l + i16
                    cols, idxs = [], []
                    for g in range(4):
                        ig = base + (256 * g)
                        cols.append(plsc.load_gather(sbuf, [ig]))
                        idxs.append(ig)
                    bv, bi = cols[0], idxs[0]
                    for g in range(1, 4):
                        cgt = cols[g] > bv
                        bv = jnp.where(cgt, cols[g], bv)
                        bi = jnp.where(cgt, idxs[g], bi)
                    key = jnp.where(bv == m, bi, 1 << 30)
                    ks, _ = plsc.sort_key_val(key, key, descending=False)
                    iw = bcast0(ks)
                    res_v = jnp.where(iota == e, m, res_v)
                    res_i = jnp.where(iota == e, iw, res_i)
                    plsc.store_scatter(sbuf, [iw], neg1, mask=lane0)
                    nb = jnp.where(idxs[0] == iw, -1.0, cols[0])
                    for g in range(1, 4):
                        nb = jnp.maximum(nb, jnp.where(idxs[g] == iw, -1.0,
                                                       cols[g]))
                    ns, _ = plsc.sort_key_val(nb, nb, descending=True)
                    m2 = bcast0(ns)
                    M = jnp.where(iota == l, m2, M)
                res_t = plsc.load_gather(tbuf, [res_i])
                obv[pl.ds(r * 16, 16)] = res_v
                obt[pl.ds(r * 16, 16)] = res_t
                return carry2

            jax.lax.fori_loop(0, _NRC, row_body, 0)
            pltpu.sync_copy(obv, ov_hbm.at[pl.ds(row0 * 16, _NRC * 16)])
            pltpu.sync_copy(obt, ot_hbm.at[pl.ds(row0 * 16, _NRC * 16)])
            return carry

        jax.lax.fori_loop(0, _NCH, chunk_body, 0)


def _dense_body(tks_ref, gs_ref, tkt_ref, gt_ref, conf_ref, out_ref):
    i = pl.program_id(0)
    conf = conf_ref[...]
    wa = jnp.sum(jnp.sum(_bce(tks_ref[...], tkt_ref[...]), axis=1) * conf)
    wb = jnp.sum(jnp.sum(_bce(gs_ref[...], gt_ref[...]), axis=1) * conf)
    z = wa * 0.0
    vals = jnp.stack([wa, wb, z, z, z, z, z, z]).reshape(1, 8)

    @pl.when(i == 0)
    def _():
        out_ref[...] = jnp.zeros_like(out_ref)

    out_ref[...] += vals


def _combine_body(tkv_ref, tkt_ref, gv_ref, gt_ref, out_ref):
    def one(vref, tref):
        v = vref[...]
        t = tref[...]
        hn_sum = jnp.zeros((_B,), jnp.float32)
        hn_cnt = jnp.zeros((_B,), jnp.float32)
        elig_seen = jnp.zeros((_B,), jnp.float32)
        for j in range(_TOPK):
            vj = v[:, j]
            tj = t[:, j]
            elig = tj < 0.5
            sel = elig & (elig_seen < _HN_K)
            hn_sum += jnp.where(sel, _bce(vj, tj), 0.0)
            hn_cnt += sel.astype(jnp.float32)
            elig_seen += elig.astype(jnp.float32)
        return jnp.sum(hn_sum), jnp.sum(hn_cnt)

    a0, a1 = one(tkv_ref, tkt_ref)
    b0, b1 = one(gv_ref, gt_ref)
    z = a0 * 0.0
    out_ref[...] = jnp.stack([a0, a1, b0, b1, z, z, z, z]).reshape(1, 8)


def kernel(tk_to_genomic_scores, genomic_to_tk_scores, tk_to_genomic_targets,
           genomic_to_tk_targets, confidences):
    tkv, tkt6, gv, gt6 = _sc_topk(tk_to_genomic_scores.reshape(-1),
                                  genomic_to_tk_scores.reshape(-1),
                                  tk_to_genomic_targets.reshape(-1),
                                  genomic_to_tk_targets.reshape(-1))
    tkv, tkt6, gv, gt6 = (x.reshape(_B, 16) for x in (tkv, tkt6, gv, gt6))

    row_spec = pl.BlockSpec((_BR, _C), lambda i: (i, 0))
    dense = pl.pallas_call(
        _dense_body,
        grid=(_NBLK,),
        in_specs=[row_spec, row_spec, row_spec, row_spec,
                  pl.BlockSpec((_BR,), lambda i: (i,))],
        out_specs=pl.BlockSpec((1, 8), lambda i: (0, 0)),
        out_shape=jax.ShapeDtypeStruct((1, 8), jnp.float32),
    )(tk_to_genomic_scores, genomic_to_tk_scores, tk_to_genomic_targets,
      genomic_to_tk_targets, confidences)[0]

    six_spec = pl.BlockSpec((_B, 16), lambda: (0, 0))
    hn = pl.pallas_call(
        _combine_body,
        in_specs=[six_spec, six_spec, six_spec, six_spec],
        out_specs=pl.BlockSpec((1, 8), lambda: (0, 0)),
        out_shape=jax.ShapeDtypeStruct((1, 8), jnp.float32),
    )(tkv, tkt6, gv, gt6)[0]

    denom = float(_B * _C)
    tk_loss = dense[0] / denom + _HN_W * hn[0] / (hn[1] + 1e-8)
    g_loss = dense[1] / denom + _HN_W * hn[2] / (hn[3] + 1e-8)
    total = _TK_W * tk_loss + _G_W * g_loss
    return (total, tk_loss, g_loss)


# speedup vs baseline: 1.2396x; 1.2396x over previous
import functools

import jax
import jax.numpy as jnp
from jax.experimental import pallas as pl
from jax.experimental.pallas import tpu as pltpu
from jax.experimental.pallas import tpu_sc as plsc

_B, _C = 4096, 1000
_CP = 1024
_BR = 256
_NBLK = _B // _BR
_TK_W, _G_W, _HN_W, _HN_K = 0.6, 0.4, 0.5, 2
_TOPK = 6

_NC, _NS = 2, 16
_NW = _NC * _NS
_RPT = _B // _NW
_NRC = 16
_NCH = _RPT // _NRC

_NLN2 = -0.6931471805599453


def _bce(p, t):
    p = jnp.clip(p, 1e-7, 1.0 - 1e-7)
    l2p = jnp.log2(p)
    l2q = jnp.log2(1.0 - p)
    return _NLN2 * (l2q + t * (l2p - l2q))


_sc_mesh = plsc.VectorSubcoreMesh(core_axis_name="c", subcore_axis_name="s")


@functools.partial(
    pl.kernel,
    mesh=_sc_mesh,
    out_type=[jax.ShapeDtypeStruct((_B * 16,), jnp.float32) for _ in range(2)],
    scratch_types=[
        pltpu.VMEM((_NRC * _CP,), jnp.float32),
        pltpu.VMEM((_NRC * _CP,), jnp.float32),
        pltpu.VMEM((_NRC * 16,), jnp.float32),
        pltpu.VMEM((_NRC * 16,), jnp.float32),
        pltpu.SemaphoreType.DMA,
    ],
    compiler_params=pltpu.CompilerParams(needs_layout_passes=False),
)
def _sc_topk(s_hbm, t_hbm, ov_hbm, ot_hbm, sbuf, tbuf, obv, obt, sem):
    wid = jax.lax.axis_index("s") * _NC + jax.lax.axis_index("c")
    iota = jax.lax.iota(jnp.int32, 16)
    i16 = iota * 16
    z16 = iota * 0
    neg1 = jnp.full((16,), -1.0, jnp.float32)
    lane0 = iota == 0

    def bcast0(x):
        return x.at[z16].get(mode="promise_in_bounds")

    for r in range(_NRC):
        sbuf[pl.ds(r * _CP + 992, 16)] = neg1
        sbuf[pl.ds(r * _CP + 1008, 16)] = neg1

    if True:
        def chunk_body(ch, carry):
            row0 = wid * _RPT + ch * _NRC
            copies = []
            for r in range(_NRC):
                copies.append(pltpu.async_copy(
                    s_hbm.at[pl.ds((row0 + r) * _C, _C)],
                    sbuf.at[pl.ds(r * _CP, _C)], sem))
                copies.append(pltpu.async_copy(
                    t_hbm.at[pl.ds((row0 + r) * _C, _C)],
                    tbuf.at[pl.ds(r * _CP, _C)], sem))
            for cp in copies:
                cp.wait()

            def row_body(r, carry2):
                rb = r * _CP
                a0 = sbuf[pl.ds(rb, 16)]
                a1 = sbuf[pl.ds(rb + 16, 16)]
                a2 = sbuf[pl.ds(rb + 32, 16)]
                a3 = sbuf[pl.ds(rb + 48, 16)]
                for v in range(4, 64, 4):
                    a0 = jnp.maximum(a0, sbuf[pl.ds(rb + v * 16, 16)])
                    a1 = jnp.maximum(a1, sbuf[pl.ds(rb + v * 16 + 16, 16)])
                    a2 = jnp.maximum(a2, sbuf[pl.ds(rb + v * 16 + 32, 16)])
                    a3 = jnp.maximum(a3, sbuf[pl.ds(rb + v * 16 + 48, 16)])
                M = jnp.maximum(jnp.maximum(a0, a1), jnp.maximum(a2, a3))
                rbv = jnp.full((16,), rb, jnp.int32)
                res_v = jnp.zeros((16,), jnp.float32)
                res_i = rbv
                for e in range(_TOPK):
                    svals, sids = plsc.sort_key_val(M, iota, descending=True)
                    m = bcast0(svals)
                    l = bcast0(sids)
                    base = rbv + l + i16
                    cols, idxs = [], []
                    for g in range(4):
                        ig = base + (256 * g)
                        cols.append(plsc.load_gather(sbuf, [ig]))
                        idxs.append(ig)
                    bv, bi = cols[0], idxs[0]
                    for g in range(1, 4):
                        cgt = cols[g] > bv
                        bv = jnp.where(cgt, cols[g], bv)
                        bi = jnp.where(cgt, idxs[g], bi)
                    key = jnp.where(bv == m, bi, 1 << 30)
                    ks, _ = plsc.sort_key_val(key, key, descending=False)
                    iw = bcast0(ks)
                    res_v = jnp.where(iota == e, m, res_v)
                    res_i = jnp.where(iota == e, iw, res_i)
                    plsc.store_scatter(sbuf, [iw], neg1, mask=lane0)
                    nb = jnp.where(idxs[0] == iw, -1.0, cols[0])
                    for g in range(1, 4):
                        nb = jnp.maximum(nb, jnp.where(idxs[g] == iw, -1.0,
                                                       cols[g]))
                    ns, _ = plsc.sort_key_val(nb, nb, descending=True)
                    m2 = bcast0(ns)
                    M = jnp.where(iota == l, m2, M)
                res_t = plsc.load_gather(tbuf, [res_i])
                obv[pl.ds(r * 16, 16)] = res_v
                obt[pl.ds(r * 16, 16)] = res_t
                return carry2

            jax.lax.fori_loop(0, _NRC, row_body, 0)
            pltpu.sync_copy(obv, ov_hbm.at[pl.ds(row0 * 16, _NRC * 16)])
            pltpu.sync_copy(obt, ot_hbm.at[pl.ds(row0 * 16, _NRC * 16)])
            return carry

        jax.lax.fori_loop(0, _NCH, chunk_body, 0)


def _dense_body(tks_ref, gs_ref, tkt_ref, gt_ref, conf_ref, out_ref):
    i = pl.program_id(0)
    conf = conf_ref[...]
    wa = jnp.sum(jnp.sum(_bce(tks_ref[...], tkt_ref[...]), axis=1) * conf)
    wb = jnp.sum(jnp.sum(_bce(gs_ref[...], gt_ref[...]), axis=1) * conf)
    z = wa * 0.0
    vals = jnp.stack([wa, wb, z, z, z, z, z, z]).reshape(1, 8)

    @pl.when(i == 0)
    def _():
        out_ref[...] = jnp.zeros_like(out_ref)

    out_ref[...] += vals


def _combine_body(tkv_ref, tkt_ref, gv_ref, gt_ref, out_ref):
    def one(vref, tref):
        v = vref[...]
        t = tref[...]
        hn_sum = jnp.zeros((_B,), jnp.float32)
        hn_cnt = jnp.zeros((_B,), jnp.float32)
        elig_seen = jnp.zeros((_B,), jnp.float32)
        for j in range(_TOPK):
            vj = v[:, j]
            tj = t[:, j]
            elig = tj < 0.5
            sel = elig & (elig_seen < _HN_K)
            hn_sum += jnp.where(sel, _bce(vj, tj), 0.0)
            hn_cnt += sel.astype(jnp.float32)
            elig_seen += elig.astype(jnp.float32)
        return jnp.sum(hn_sum), jnp.sum(hn_cnt)

    a0, a1 = one(tkv_ref, tkt_ref)
    b0, b1 = one(gv_ref, gt_ref)
    z = a0 * 0.0
    out_ref[...] = jnp.stack([a0, a1, b0, b1, z, z, z, z]).reshape(1, 8)


def kernel(tk_to_genomic_scores, genomic_to_tk_scores, tk_to_genomic_targets,
           genomic_to_tk_targets, confidences):
    tkv, tkt6 = _sc_topk(tk_to_genomic_scores.reshape(-1),
                         tk_to_genomic_targets.reshape(-1))
    gv, gt6 = _sc_topk(genomic_to_tk_scores.reshape(-1),
                       genomic_to_tk_targets.reshape(-1))
    tkv, tkt6, gv, gt6 = (x.reshape(_B, 16) for x in (tkv, tkt6, gv, gt6))

    row_spec = pl.BlockSpec((_BR, _C), lambda i: (i, 0))
    dense = pl.pallas_call(
        _dense_body,
        grid=(_NBLK,),
        in_specs=[row_spec, row_spec, row_spec, row_spec,
                  pl.BlockSpec((_BR,), lambda i: (i,))],
        out_specs=pl.BlockSpec((1, 8), lambda i: (0, 0)),
        out_shape=jax.ShapeDtypeStruct((1, 8), jnp.float32),
    )(tk_to_genomic_scores, genomic_to_tk_scores, tk_to_genomic_targets,
      genomic_to_tk_targets, confidences)[0]

    six_spec = pl.BlockSpec((_B, 16), lambda: (0, 0))
    hn = pl.pallas_call(
        _combine_body,
        in_specs=[six_spec, six_spec, six_spec, six_spec],
        out_specs=pl.BlockSpec((1, 8), lambda: (0, 0)),
        out_shape=jax.ShapeDtypeStruct((1, 8), jnp.float32),
    )(tkv, tkt6, gv, gt6)[0]

    denom = float(_B * _C)
    tk_loss = dense[0] / denom + _HN_W * hn[0] / (hn[1] + 1e-8)
    g_loss = dense[1] / denom + _HN_W * hn[2] / (hn[3] + 1e-8)
    total = _TK_W * tk_loss + _G_W * g_loss
    return (total, tk_loss, g_loss)


# speedup vs baseline: 1.3022x; 1.0505x over previous
import functools

import jax
import jax.numpy as jnp
from jax.experimental import pallas as pl
from jax.experimental.pallas import tpu as pltpu
from jax.experimental.pallas import tpu_sc as plsc

_B, _C = 4096, 1000
_CP = 1024
_BR = 256
_NBLK = _B // _BR
_TK_W, _G_W, _HN_W, _HN_K = 0.6, 0.4, 0.5, 2
_TOPK = 6

_NC, _NS = 2, 16
_NW = _NC * _NS
_RPT = _B // _NW
_NRC = 16
_NCH = _RPT // _NRC

_NLN2 = -0.6931471805599453


def _bce(p, t):
    p = jnp.clip(p, 1e-7, 1.0 - 1e-7)
    l2p = jnp.log2(p)
    l2q = jnp.log2(1.0 - p)
    return _NLN2 * (l2q + t * (l2p - l2q))


_sc_mesh = plsc.VectorSubcoreMesh(core_axis_name="c", subcore_axis_name="s")


@functools.partial(
    pl.kernel,
    mesh=_sc_mesh,
    out_type=[jax.ShapeDtypeStruct((_B * 16,), jnp.float32) for _ in range(2)],
    scratch_types=[
        pltpu.VMEM((_NRC * _CP,), jnp.float32),
        pltpu.VMEM((_NRC * _CP,), jnp.float32),
        pltpu.VMEM((_NRC * _CP,), jnp.float32),
        pltpu.VMEM((_NRC * _CP,), jnp.float32),
        pltpu.VMEM((_NRC * 16,), jnp.float32),
        pltpu.VMEM((_NRC * 16,), jnp.float32),
        pltpu.SemaphoreType.DMA,
        pltpu.SemaphoreType.DMA,
    ],
    compiler_params=pltpu.CompilerParams(needs_layout_passes=False),
)
def _sc_topk(s_hbm, t_hbm, ov_hbm, ot_hbm, sbuf0, tbuf0, sbuf1, tbuf1,
             obv, obt, sem0, sem1):
    wid = jax.lax.axis_index("s") * _NC + jax.lax.axis_index("c")
    iota = jax.lax.iota(jnp.int32, 16)
    i16 = iota * 16
    z16 = iota * 0
    neg1 = jnp.full((16,), -1.0, jnp.float32)
    lane0 = iota == 0

    def bcast0(x):
        return x.at[z16].get(mode="promise_in_bounds")

    for r in range(_NRC):
        for sb in (sbuf0, sbuf1):
            sb[pl.ds(r * _CP + 992, 16)] = neg1
            sb[pl.ds(r * _CP + 1008, 16)] = neg1

    def issue(ch, sb, tb, sem):
        row0 = wid * _RPT + ch * _NRC
        for r in range(_NRC):
            pltpu.async_copy(s_hbm.at[pl.ds((row0 + r) * _C, _C)],
                             sb.at[pl.ds(r * _CP, _C)], sem)
            pltpu.async_copy(t_hbm.at[pl.ds((row0 + r) * _C, _C)],
                             tb.at[pl.ds(r * _CP, _C)], sem)

    def drain(ch, sb, tb, sem):
        row0 = wid * _RPT + ch * _NRC
        for r in range(_NRC):
            pltpu.make_async_copy(s_hbm.at[pl.ds((row0 + r) * _C, _C)],
                                  sb.at[pl.ds(r * _CP, _C)], sem).wait()
            pltpu.make_async_copy(t_hbm.at[pl.ds((row0 + r) * _C, _C)],
                                  tb.at[pl.ds(r * _CP, _C)], sem).wait()

    def process_row(sb, tb, r):
        rb = r * _CP
        a0 = sb[pl.ds(rb, 16)]
        a1 = sb[pl.ds(rb + 16, 16)]
        a2 = sb[pl.ds(rb + 32, 16)]
        a3 = sb[pl.ds(rb + 48, 16)]
        for v in range(4, 64, 4):
            a0 = jnp.maximum(a0, sb[pl.ds(rb + v * 16, 16)])
            a1 = jnp.maximum(a1, sb[pl.ds(rb + v * 16 + 16, 16)])
            a2 = jnp.maximum(a2, sb[pl.ds(rb + v * 16 + 32, 16)])
            a3 = jnp.maximum(a3, sb[pl.ds(rb + v * 16 + 48, 16)])
        M = jnp.maximum(jnp.maximum(a0, a1), jnp.maximum(a2, a3))
        rbv = jnp.full((16,), rb, jnp.int32)
        res_v = jnp.zeros((16,), jnp.float32)
        res_i = rbv
        for e in range(_TOPK):
            svals, sids = plsc.sort_key_val(M, iota, descending=True)
            m = bcast0(svals)
            l = bcast0(sids)
            base = rbv + l + i16
            cols, idxs = [], []
            for g in range(4):
                ig = base + (256 * g)
                cols.append(plsc.load_gather(sb, [ig]))
                idxs.append(ig)
            bv, bi = cols[0], idxs[0]
            for g in range(1, 4):
                cgt = cols[g] > bv
                bv = jnp.where(cgt, cols[g], bv)
                bi = jnp.where(cgt, idxs[g], bi)
            key = jnp.where(bv == m, bi, 1 << 30)
            ks, _ = plsc.sort_key_val(key, key, descending=False)
            iw = bcast0(ks)
            res_v = jnp.where(iota == e, m, res_v)
            res_i = jnp.where(iota == e, iw, res_i)
            plsc.store_scatter(sb, [iw], neg1, mask=lane0)
            nb = jnp.where(idxs[0] == iw, -1.0, cols[0])
            for g in range(1, 4):
                nb = jnp.maximum(nb, jnp.where(idxs[g] == iw, -1.0, cols[g]))
            ns, _ = plsc.sort_key_val(nb, nb, descending=True)
            m2 = bcast0(ns)
            M = jnp.where(iota == l, m2, M)
        res_t = plsc.load_gather(tb, [res_i])
        obv[pl.ds(r * 16, 16)] = res_v
        obt[pl.ds(r * 16, 16)] = res_t

    def process(ch, sb, tb):
        def pair_body(j, c):
            process_row(sb, tb, 2 * j)
            process_row(sb, tb, 2 * j + 1)
            return c
        jax.lax.fori_loop(0, _NRC // 2, pair_body, 0)
        row0 = wid * _RPT + ch * _NRC
        pltpu.sync_copy(obv, ov_hbm.at[pl.ds(row0 * 16, _NRC * 16)])
        pltpu.sync_copy(obt, ot_hbm.at[pl.ds(row0 * 16, _NRC * 16)])

    issue(0, sbuf0, tbuf0, sem0)

    def outer(i, c):
        ch0 = 2 * i
        drain(ch0, sbuf0, tbuf0, sem0)
        issue(ch0 + 1, sbuf1, tbuf1, sem1)
        process(ch0, sbuf0, tbuf0)

        @pl.when(i < _NCH // 2 - 1)
        def _():
            issue(ch0 + 2, sbuf0, tbuf0, sem0)

        drain(ch0 + 1, sbuf1, tbuf1, sem1)
        process(ch0 + 1, sbuf1, tbuf1)
        return c

    jax.lax.fori_loop(0, _NCH // 2, outer, 0)


def _dense_body(tks_ref, gs_ref, tkt_ref, gt_ref, conf_ref, out_ref):
    i = pl.program_id(0)
    conf = conf_ref[...]
    wa = jnp.sum(jnp.sum(_bce(tks_ref[...], tkt_ref[...]), axis=1) * conf)
    wb = jnp.sum(jnp.sum(_bce(gs_ref[...], gt_ref[...]), axis=1) * conf)
    z = wa * 0.0
    vals = jnp.stack([wa, wb, z, z, z, z, z, z]).reshape(1, 8)

    @pl.when(i == 0)
    def _():
        out_ref[...] = jnp.zeros_like(out_ref)

    out_ref[...] += vals


def _combine_body(tkv_ref, tkt_ref, gv_ref, gt_ref, out_ref):
    def one(vref, tref):
        v = vref[...]
        t = tref[...]
        hn_sum = jnp.zeros((_B,), jnp.float32)
        hn_cnt = jnp.zeros((_B,), jnp.float32)
        elig_seen = jnp.zeros((_B,), jnp.float32)
        for j in range(_TOPK):
            vj = v[:, j]
            tj = t[:, j]
            elig = tj < 0.5
            sel = elig & (elig_seen < _HN_K)
            hn_sum += jnp.where(sel, _bce(vj, tj), 0.0)
            hn_cnt += sel.astype(jnp.float32)
            elig_seen += elig.astype(jnp.float32)
        return jnp.sum(hn_sum), jnp.sum(hn_cnt)

    a0, a1 = one(tkv_ref, tkt_ref)
    b0, b1 = one(gv_ref, gt_ref)
    z = a0 * 0.0
    out_ref[...] = jnp.stack([a0, a1, b0, b1, z, z, z, z]).reshape(1, 8)


def kernel(tk_to_genomic_scores, genomic_to_tk_scores, tk_to_genomic_targets,
           genomic_to_tk_targets, confidences):
    tkv, tkt6 = _sc_topk(tk_to_genomic_scores.reshape(-1),
                         tk_to_genomic_targets.reshape(-1))
    gv, gt6 = _sc_topk(genomic_to_tk_scores.reshape(-1),
                       genomic_to_tk_targets.reshape(-1))
    tkv, tkt6, gv, gt6 = (x.reshape(_B, 16) for x in (tkv, tkt6, gv, gt6))

    row_spec = pl.BlockSpec((_BR, _C), lambda i: (i, 0))
    dense = pl.pallas_call(
        _dense_body,
        grid=(_NBLK,),
        in_specs=[row_spec, row_spec, row_spec, row_spec,
                  pl.BlockSpec((_BR,), lambda i: (i,))],
        out_specs=pl.BlockSpec((1, 8), lambda i: (0, 0)),
        out_shape=jax.ShapeDtypeStruct((1, 8), jnp.float32),
    )(tk_to_genomic_scores, genomic_to_tk_scores, tk_to_genomic_targets,
      genomic_to_tk_targets, confidences)[0]

    six_spec = pl.BlockSpec((_B, 16), lambda: (0, 0))
    hn = pl.pallas_call(
        _combine_body,
        in_specs=[six_spec, six_spec, six_spec, six_spec],
        out_specs=pl.BlockSpec((1, 8), lambda: (0, 0)),
        out_shape=jax.ShapeDtypeStruct((1, 8), jnp.float32),
    )(tkv, tkt6, gv, gt6)[0]

    denom = float(_B * _C)
    tk_loss = dense[0] / denom + _HN_W * hn[0] / (hn[1] + 1e-8)
    g_loss = dense[1] / denom + _HN_W * hn[2] / (hn[3] + 1e-8)
    total = _TK_W * tk_loss + _G_W * g_loss
    return (total, tk_loss, g_loss)


# speedup vs baseline: 1.3061x; 1.0030x over previous
import functools

import jax
import jax.numpy as jnp
from jax.experimental import pallas as pl
from jax.experimental.pallas import tpu as pltpu
from jax.experimental.pallas import tpu_sc as plsc

_B, _C = 4096, 1000
_CP = 1024
_BR = 256
_NBLK = _B // _BR
_TK_W, _G_W, _HN_W, _HN_K = 0.6, 0.4, 0.5, 2
_TOPK = 6

_NC, _NS = 2, 16
_NW = _NC * _NS
_RPT = _B // _NW
_NRC = 16
_NCH = _RPT // _NRC

_NLN2 = -0.6931471805599453


def _bce(p, t):
    p = jnp.clip(p, 1e-7, 1.0 - 1e-7)
    l2p = jnp.log2(p)
    l2q = jnp.log2(1.0 - p)
    return _NLN2 * (l2q + t * (l2p - l2q))


_sc_mesh = plsc.VectorSubcoreMesh(core_axis_name="c", subcore_axis_name="s")


@functools.partial(
    pl.kernel,
    mesh=_sc_mesh,
    out_type=[jax.ShapeDtypeStruct((_B * 16,), jnp.float32) for _ in range(2)],
    scratch_types=[
        pltpu.VMEM((_NRC * _CP,), jnp.float32),
        pltpu.VMEM((_NRC * _CP,), jnp.float32),
        pltpu.VMEM((_NRC * _CP,), jnp.float32),
        pltpu.VMEM((_NRC * _CP,), jnp.float32),
        pltpu.VMEM((_NRC * 16,), jnp.float32),
        pltpu.VMEM((_NRC * 16,), jnp.float32),
        pltpu.SemaphoreType.DMA,
        pltpu.SemaphoreType.DMA,
    ],
    compiler_params=pltpu.CompilerParams(needs_layout_passes=False),
)
def _sc_topk(s_hbm, t_hbm, ov_hbm, ot_hbm, sbuf0, tbuf0, sbuf1, tbuf1,
             obv, obt, sem0, sem1):
    wid = jax.lax.axis_index("s") * _NC + jax.lax.axis_index("c")
    iota = jax.lax.iota(jnp.int32, 16)
    i16 = iota * 16
    z16 = iota * 0
    neg1 = jnp.full((16,), -1.0, jnp.float32)
    lane0 = iota == 0

    def bcast0(x):
        return x.at[z16].get(mode="promise_in_bounds")

    for r in range(_NRC):
        for sb in (sbuf0, sbuf1):
            sb[pl.ds(r * _CP + 992, 16)] = neg1
            sb[pl.ds(r * _CP + 1008, 16)] = neg1

    def issue(ch, sb, tb, sem):
        row0 = wid * _RPT + ch * _NRC
        for r in range(_NRC):
            pltpu.async_copy(s_hbm.at[pl.ds((row0 + r) * _C, _C)],
                             sb.at[pl.ds(r * _CP, _C)], sem)
            pltpu.async_copy(t_hbm.at[pl.ds((row0 + r) * _C, _C)],
                             tb.at[pl.ds(r * _CP, _C)], sem)

    def drain(ch, sb, tb, sem):
        row0 = wid * _RPT + ch * _NRC
        for r in range(_NRC):
            pltpu.make_async_copy(s_hbm.at[pl.ds((row0 + r) * _C, _C)],
                                  sb.at[pl.ds(r * _CP, _C)], sem).wait()
            pltpu.make_async_copy(t_hbm.at[pl.ds((row0 + r) * _C, _C)],
                                  tb.at[pl.ds(r * _CP, _C)], sem).wait()

    def process_row(sb, tb, r):
        rb = r * _CP
        a0 = sb[pl.ds(rb, 16)]
        a1 = sb[pl.ds(rb + 16, 16)]
        a2 = sb[pl.ds(rb + 32, 16)]
        a3 = sb[pl.ds(rb + 48, 16)]
        for v in range(4, 64, 4):
            a0 = jnp.maximum(a0, sb[pl.ds(rb + v * 16, 16)])
            a1 = jnp.maximum(a1, sb[pl.ds(rb + v * 16 + 16, 16)])
            a2 = jnp.maximum(a2, sb[pl.ds(rb + v * 16 + 32, 16)])
            a3 = jnp.maximum(a3, sb[pl.ds(rb + v * 16 + 48, 16)])
        M = jnp.maximum(jnp.maximum(a0, a1), jnp.maximum(a2, a3))
        rbv = jnp.full((16,), rb, jnp.int32)
        res_v = jnp.zeros((16,), jnp.float32)
        res_i = rbv
        for e in range(_TOPK):
            svals, sids = plsc.sort_key_val(M, iota, descending=True)
            m = bcast0(svals)
            l = bcast0(sids)
            base = rbv + l + i16
            cols, idxs = [], []
            for g in range(4):
                ig = base + (256 * g)
                cols.append(plsc.load_gather(sb, [ig]))
                idxs.append(ig)
            bv, bi = cols[0], idxs[0]
            for g in range(1, 4):
                cgt = cols[g] > bv
                bv = jnp.where(cgt, cols[g], bv)
                bi = jnp.where(cgt, idxs[g], bi)
            key = jnp.where(bv == m, bi, 1 << 30)
            ks, _ = plsc.sort_key_val(key, key, descending=False)
            iw = bcast0(ks)
            res_v = jnp.where(iota == e, m, res_v)
            res_i = jnp.where(iota == e, iw, res_i)
            plsc.store_scatter(sb, [iw], neg1, mask=lane0)
            nb = jnp.where(idxs[0] == iw, -1.0, cols[0])
            for g in range(1, 4):
                nb = jnp.maximum(nb, jnp.where(idxs[g] == iw, -1.0, cols[g]))
            ns, _ = plsc.sort_key_val(nb, nb, descending=True)
            m2 = bcast0(ns)
            M = jnp.where(iota == l, m2, M)
        res_t = plsc.load_gather(tb, [res_i])
        obv[pl.ds(r * 16, 16)] = res_v
        obt[pl.ds(r * 16, 16)] = res_t

    def process(ch, sb, tb):
        def pair_body(j, c):
            process_row(sb, tb, 2 * j)
            process_row(sb, tb, 2 * j + 1)
            return c
        jax.lax.fori_loop(0, _NRC // 2, pair_body, 0)
        row0 = wid * _RPT + ch * _NRC
        pltpu.sync_copy(obv, ov_hbm.at[pl.ds(row0 * 16, _NRC * 16)])
        pltpu.sync_copy(obt, ot_hbm.at[pl.ds(row0 * 16, _NRC * 16)])

    issue(0, sbuf0, tbuf0, sem0)

    def outer(i, c):
        ch0 = 2 * i
        drain(ch0, sbuf0, tbuf0, sem0)
        issue(ch0 + 1, sbuf1, tbuf1, sem1)
        process(ch0, sbuf0, tbuf0)
        issue(jax.lax.rem(ch0 + 2, _NCH), sbuf0, tbuf0, sem0)
        drain(ch0 + 1, sbuf1, tbuf1, sem1)
        process(ch0 + 1, sbuf1, tbuf1)
        return c

    jax.lax.fori_loop(0, _NCH // 2, outer, 0)
    drain(0, sbuf0, tbuf0, sem0)


def _dense_body(tks_ref, gs_ref, tkt_ref, gt_ref, conf_ref, out_ref):
    i = pl.program_id(0)
    conf = conf_ref[...]
    wa = jnp.sum(jnp.sum(_bce(tks_ref[...], tkt_ref[...]), axis=1) * conf)
    wb = jnp.sum(jnp.sum(_bce(gs_ref[...], gt_ref[...]), axis=1) * conf)
    z = wa * 0.0
    vals = jnp.stack([wa, wb, z, z, z, z, z, z]).reshape(1, 8)

    @pl.when(i == 0)
    def _():
        out_ref[...] = jnp.zeros_like(out_ref)

    out_ref[...] += vals


def _combine_body(tkv_ref, tkt_ref, gv_ref, gt_ref, out_ref):
    def one(vref, tref):
        v = vref[...]
        t = tref[...]
        hn_sum = jnp.zeros((_B,), jnp.float32)
        hn_cnt = jnp.zeros((_B,), jnp.float32)
        elig_seen = jnp.zeros((_B,), jnp.float32)
        for j in range(_TOPK):
            vj = v[:, j]
            tj = t[:, j]
            elig = tj < 0.5
            sel = elig & (elig_seen < _HN_K)
            hn_sum += jnp.where(sel, _bce(vj, tj), 0.0)
            hn_cnt += sel.astype(jnp.float32)
            elig_seen += elig.astype(jnp.float32)
        return jnp.sum(hn_sum), jnp.sum(hn_cnt)

    a0, a1 = one(tkv_ref, tkt_ref)
    b0, b1 = one(gv_ref, gt_ref)
    z = a0 * 0.0
    out_ref[...] = jnp.stack([a0, a1, b0, b1, z, z, z, z]).reshape(1, 8)


def kernel(tk_to_genomic_scores, genomic_to_tk_scores, tk_to_genomic_targets,
           genomic_to_tk_targets, confidences):
    tkv, tkt6 = _sc_topk(tk_to_genomic_scores.reshape(-1),
                         tk_to_genomic_targets.reshape(-1))
    gv, gt6 = _sc_topk(genomic_to_tk_scores.reshape(-1),
                       genomic_to_tk_targets.reshape(-1))
    tkv, tkt6, gv, gt6 = (x.reshape(_B, 16) for x in (tkv, tkt6, gv, gt6))

    row_spec = pl.BlockSpec((_BR, _C), lambda i: (i, 0))
    dense = pl.pallas_call(
        _dense_body,
        grid=(_NBLK,),
        in_specs=[row_spec, row_spec, row_spec, row_spec,
                  pl.BlockSpec((_BR,), lambda i: (i,))],
        out_specs=pl.BlockSpec((1, 8), lambda i: (0, 0)),
        out_shape=jax.ShapeDtypeStruct((1, 8), jnp.float32),
    )(tk_to_genomic_scores, genomic_to_tk_scores, tk_to_genomic_targets,
      genomic_to_tk_targets, confidences)[0]

    six_spec = pl.BlockSpec((_B, 16), lambda: (0, 0))
    hn = pl.pallas_call(
        _combine_body,
        in_specs=[six_spec, six_spec, six_spec, six_spec],
        out_specs=pl.BlockSpec((1, 8), lambda: (0, 0)),
        out_shape=jax.ShapeDtypeStruct((1, 8), jnp.float32),
    )(tkv, tkt6, gv, gt6)[0]

    denom = float(_B * _C)
    tk_loss = dense[0] / denom + _HN_W * hn[0] / (hn[1] + 1e-8)
    g_loss = dense[1] / denom + _HN_W * hn[2] / (hn[3] + 1e-8)
    total = _TK_W * tk_loss + _G_W * g_loss
    return (total, tk_loss, g_loss)
